# baseline (speedup 1.0000x reference)
import functools

import jax
import jax.numpy as jnp
from jax import lax
from jax.experimental import pallas as pl
from jax.experimental.pallas import tpu as pltpu
from jax.experimental.pallas import tpu_sc as plsc

_MIN_NORM = 1e-15
_BALL_EPS = 4e-3
_KPAD = 1024
_BB = 512
_NW = 32
_NSEM = 4


def _tanh(x):
    return jnp.tanh(jnp.clip(x, -15.0, 15.0))


def _tc_body(idx_ref, head_ref, tail_ref, tab_ref, out_ref):
    f32 = jnp.float32
    idx = idx_ref[0:1, :]
    x0 = head_ref[...]
    tab = tab_ref[...]
    D, bB = x0.shape
    K = tab.shape[1]

    kio = lax.broadcasted_iota(jnp.int32, (K, bB), 0)
    onehot = (kio == idx).astype(f32)
    g = jnp.dot(tab, onehot, preferred_element_type=f32)
    rel_half = g[:D, :]
    gd = g[D:2 * D, :]
    c_raw = g[2 * D:2 * D + 1, :]
    c = jnp.maximum(c_raw, 0.0) + jnp.log(1.0 + jnp.exp(-jnp.abs(c_raw)))
    sqrt_c = jnp.sqrt(c)

    r_io = lax.broadcasted_iota(jnp.int32, (D, D), 0)
    c_io = lax.broadcasted_iota(jnp.int32, (D, D), 1)
    P = (r_io == (c_io ^ 1)).astype(f32)

    def psw(v):
        return jnp.dot(P, v, preferred_element_type=f32)

    row = lax.broadcasted_iota(jnp.int32, (D, bB), 0)
    even = (row & 1) == 0
    gs = psw(gd)
    nrm = jnp.sqrt(gd * gd + gs * gs)
    gn = gd / jnp.maximum(nrm, _MIN_NORM)
    gns = psw(gn)
    xs = psw(x0)
    refl = jnp.where(even, gn * x0 + gns * xs, gn * xs - gns * x0)

    def _project(x):
        n = jnp.maximum(jnp.sqrt(jnp.sum(x * x, 0, keepdims=True)), _MIN_NORM)
        maxn = (1.0 - _BALL_EPS) / sqrt_c
        return jnp.where(n > maxn, x / n * maxn, x)

    def _expmap0(u):
        un = jnp.maximum(jnp.sqrt(jnp.sum(u * u, 0, keepdims=True)), _MIN_NORM)
        gmm = _tanh(sqrt_c * un) * u / (sqrt_c * un)
        return _project(gmm)

    lhs = _expmap0(refl)
    rel_h = _expmap0(rel_half)

    x2m = jnp.sum(lhs * lhs, 0, keepdims=True)
    y2m = jnp.sum(rel_h * rel_h, 0, keepdims=True)
    xym = jnp.sum(lhs * rel_h, 0, keepdims=True)
    num = (1.0 + 2.0 * c * xym + c * y2m) * lhs + (1.0 - c * x2m) * rel_h
    den = 1.0 + 2.0 * c * xym + c * c * x2m * y2m
    res = _project(num / jnp.maximum(den, _MIN_NORM))

    tail = tail_ref[...]
    v2 = jnp.sum(tail * tail, axis=1)
    vnorm = jnp.sqrt(v2)
    xv = jnp.sum(res[None, :, :] * tail, axis=1) / vnorm
    x2 = jnp.sum(res * res, 0, keepdims=True)
    gam = _tanh(sqrt_c * vnorm) / sqrt_c
    c1 = 1.0 - 2.0 * c * gam * xv + c * gam * gam
    c2 = 1.0 - c * x2
    numd = jnp.sqrt(jnp.maximum(
        c1 * c1 * x2 + c2 * c2 * gam * gam - 2.0 * c1 * c2 * gam * xv, 0.0))
    dend = 1.0 - 2.0 * c * gam * xv + (c * c) * (gam * gam) * x2
    pn = numd / jnp.maximum(dend, _MIN_NORM)
    sp = jnp.clip(sqrt_c * pn, -1.0 + 1e-5, 1.0 - 1e-5)
    dist = 0.5 * (jnp.log(1.0 + sp) - jnp.log(1.0 - sp))
    d = 2.0 * dist / sqrt_c
    out_ref[...] = -(d * d)


def _tc_scores(idx8, headT, tailT, tabT, interpret=False):
    NEG, D, B = tailT.shape
    return pl.pallas_call(
        _tc_body,
        grid=(B // _BB,),
        in_specs=[
            pl.BlockSpec((8, _BB), lambda i: (0, i)),
            pl.BlockSpec((D, _BB), lambda i: (0, i)),
            pl.BlockSpec((NEG, D, _BB), lambda i: (0, 0, i)),
            pl.BlockSpec((3 * D, _KPAD), lambda i: (0, 0)),
        ],
        out_specs=pl.BlockSpec((NEG, _BB), lambda i: (0, i)),
        out_shape=jax.ShapeDtypeStruct((NEG, B), jnp.float32),
        interpret=interpret,
    )(idx8, headT, tailT, tabT)


def _sc_gather_bt(bt_flat, idx_bt2d):
    n_rows, chunk = idx_bt2d.shape
    ck_w = n_rows // _NW
    n_w = ck_w * chunk
    mesh = plsc.VectorSubcoreMesh(core_axis_name="c", subcore_axis_name="s")

    @functools.partial(
        pl.kernel,
        mesh=mesh,
        out_type=jax.ShapeDtypeStruct((n_rows * chunk,), jnp.float32),
        scratch_types=[
            pltpu.VMEM((ck_w, chunk), jnp.int32),
            pltpu.VMEM((n_w,), jnp.float32),
            [pltpu.SemaphoreType.DMA] * _NSEM,
        ],
    )
    def run(bt_h, ibt_h, out_h, ibt_v, vbt_v, sems):
        wid = lax.axis_index("s") * 2 + lax.axis_index("c")
        pltpu.sync_copy(ibt_h.at[pl.ds(wid * ck_w, ck_w)], ibt_v)

        def fire(k, carry):
            off = pl.multiple_of(k * chunk, chunk)
            dst = vbt_v.at[pl.ds(off, chunk)]
            for s in range(_NSEM):
                @pl.when(k % _NSEM == s)
                def _():
                    pltpu.async_copy(bt_h.at[ibt_v.at[k]], dst, sems[s])
            return carry

        lax.fori_loop(0, ck_w, fire, 0)

        def drain(k, carry):
            off = pl.multiple_of(k * chunk, chunk)
            dst = vbt_v.at[pl.ds(off, chunk)]
            for s in range(_NSEM):
                @pl.when(k % _NSEM == s)
                def _():
                    pltpu.make_async_copy(bt_h.at[ibt_v.at[k]], dst, sems[s]).wait()
            return carry

        lax.fori_loop(0, ck_w, drain, 0)
        pltpu.sync_copy(vbt_v, out_h.at[pl.ds(wid * n_w, n_w)])

    return run(bt_flat, idx_bt2d)


def _sc_gather_bh(bh_flat, idx_bh):
    B = idx_bh.shape[0]
    rows_w = B // _NW
    mesh = plsc.VectorSubcoreMesh(core_axis_name="c", subcore_axis_name="s")

    @functools.partial(
        pl.kernel,
        mesh=mesh,
        out_type=jax.ShapeDtypeStruct((B,), jnp.float32),
        scratch_types=[
            pltpu.VMEM((rows_w,), jnp.int32),
            pltpu.VMEM((rows_w,), jnp.float32),
            pltpu.SemaphoreType.DMA,
        ],
    )
    def run(bh_h, ibh_h, outh_h, ibh_v, vbh_v, semh):
        wid = lax.axis_index("s") * 2 + lax.axis_index("c")
        pltpu.sync_copy(ibh_h.at[pl.ds(wid * rows_w, rows_w)], ibh_v)
        pltpu.async_copy(bh_h.at[ibh_v], vbh_v, semh).wait()
        pltpu.sync_copy(vbh_v, outh_h.at[pl.ds(wid * rows_w, rows_w)])

    return run(bh_flat, idx_bh)


def kernel(head, relation, tail, positive_sample, negative_sample, bh, bt,
           rel_diag, rel, c_param):
    del relation
    B, NEG, D = tail.shape
    nrel = rel.shape[0]
    f32 = jnp.float32

    headT = jnp.transpose(head.reshape(B, D).astype(f32))
    tailT = jnp.transpose(tail.astype(f32), (1, 2, 0))
    rel_idx = positive_sample[:, 1].astype(jnp.int32).reshape(1, B)
    idx8 = jnp.broadcast_to(rel_idx, (8, B))

    tab = jnp.concatenate(
        [rel[:, :D].astype(f32), rel_diag.astype(f32), c_param.astype(f32),
         jnp.zeros((nrel, D - 1), f32)], axis=1)
    tabT = jnp.transpose(jnp.pad(tab, ((0, _KPAD - nrel), (0, 0))))

    idx_bt = jnp.concatenate(
        [positive_sample[:, 2:3], negative_sample], axis=1).astype(jnp.int32)
    idx_bt2d = idx_bt.reshape(-1, 128)
    idx_bh = positive_sample[:, 0].astype(jnp.int32)

    bias_bt = _sc_gather_bt(bt.reshape(-1).astype(f32), idx_bt2d)
    bias_bh = _sc_gather_bh(bh.reshape(-1).astype(f32), idx_bh)
    scores = _tc_scores(idx8, headT, tailT, tabT)
    out = (jnp.transpose(scores) + bias_bh.reshape(B, 1)).reshape(B * NEG, 1)
    return out + bias_bt.reshape(B * NEG, 1)


# speedup vs baseline: 1.0042x; 1.0042x over previous
import functools

import jax
import jax.numpy as jnp
from jax import lax
from jax.experimental import pallas as pl
from jax.experimental.pallas import tpu as pltpu
from jax.experimental.pallas import tpu_sc as plsc

_MIN_NORM = 1e-15
_BALL_EPS = 4e-3
_KPAD = 1024
_BB = 512
_NW = 32
_NSEM = 4


def _tanh(x):
    return jnp.tanh(jnp.clip(x, -15.0, 15.0))


def _tc_body(idx_ref, head_ref, tail_ref, tab_ref, out_ref):
    f32 = jnp.float32
    idx = idx_ref[0:1, :]
    x0 = head_ref[...]
    tab = tab_ref[...]
    D, bB = x0.shape
    K = tab.shape[1]

    kio = lax.broadcasted_iota(jnp.int32, (K, bB), 0)
    onehot = (kio == idx).astype(f32)
    g = jnp.dot(tab, onehot, preferred_element_type=f32)
    rel_half = g[:D, :]
    gd = g[D:2 * D, :]
    c_raw = g[2 * D:2 * D + 1, :]
    c = jnp.maximum(c_raw, 0.0) + jnp.log(1.0 + jnp.exp(-jnp.abs(c_raw)))
    sqrt_c = jnp.sqrt(c)

    r_io = lax.broadcasted_iota(jnp.int32, (D, D), 0)
    c_io = lax.broadcasted_iota(jnp.int32, (D, D), 1)
    P = (r_io == (c_io ^ 1)).astype(f32)

    def psw(v):
        return jnp.dot(P, v, preferred_element_type=f32)

    row = lax.broadcasted_iota(jnp.int32, (D, bB), 0)
    even = (row & 1) == 0
    gs = psw(gd)
    nrm = jnp.sqrt(gd * gd + gs * gs)
    gn = gd / jnp.maximum(nrm, _MIN_NORM)
    gns = psw(gn)
    xs = psw(x0)
    refl = jnp.where(even, gn * x0 + gns * xs, gn * xs - gns * x0)

    def _project(x):
        n = jnp.maximum(jnp.sqrt(jnp.sum(x * x, 0, keepdims=True)), _MIN_NORM)
        maxn = (1.0 - _BALL_EPS) / sqrt_c
        return jnp.where(n > maxn, x / n * maxn, x)

    def _expmap0(u):
        un = jnp.maximum(jnp.sqrt(jnp.sum(u * u, 0, keepdims=True)), _MIN_NORM)
        gmm = _tanh(sqrt_c * un) * u / (sqrt_c * un)
        return _project(gmm)

    lhs = _expmap0(refl)
    rel_h = _expmap0(rel_half)

    x2m = jnp.sum(lhs * lhs, 0, keepdims=True)
    y2m = jnp.sum(rel_h * rel_h, 0, keepdims=True)
    xym = jnp.sum(lhs * rel_h, 0, keepdims=True)
    num = (1.0 + 2.0 * c * xym + c * y2m) * lhs + (1.0 - c * x2m) * rel_h
    den = 1.0 + 2.0 * c * xym + c * c * x2m * y2m
    res = _project(num / jnp.maximum(den, _MIN_NORM))

    tail = tail_ref[...]
    v2 = jnp.sum(tail * tail, axis=1)
    vnorm = jnp.sqrt(v2)
    xv = jnp.sum(res[None, :, :] * tail, axis=1) / vnorm
    x2 = jnp.sum(res * res, 0, keepdims=True)
    gam = _tanh(sqrt_c * vnorm) / sqrt_c
    c1 = 1.0 - 2.0 * c * gam * xv + c * gam * gam
    c2 = 1.0 - c * x2
    numd = jnp.sqrt(jnp.maximum(
        c1 * c1 * x2 + c2 * c2 * gam * gam - 2.0 * c1 * c2 * gam * xv, 0.0))
    dend = 1.0 - 2.0 * c * gam * xv + (c * c) * (gam * gam) * x2
    pn = numd / jnp.maximum(dend, _MIN_NORM)
    sp = jnp.clip(sqrt_c * pn, -1.0 + 1e-5, 1.0 - 1e-5)
    dist = 0.5 * (jnp.log(1.0 + sp) - jnp.log(1.0 - sp))
    d = 2.0 * dist / sqrt_c
    out_ref[...] = -(d * d)


def _tc_scores(idx8, headT, tailT, tabT, interpret=False):
    NEG, D, B = tailT.shape
    return pl.pallas_call(
        _tc_body,
        grid=(B // _BB,),
        in_specs=[
            pl.BlockSpec((8, _BB), lambda i: (0, i)),
            pl.BlockSpec((D, _BB), lambda i: (0, i)),
            pl.BlockSpec((NEG, D, _BB), lambda i: (0, 0, i)),
            pl.BlockSpec((3 * D, _KPAD), lambda i: (0, 0)),
        ],
        out_specs=pl.BlockSpec((NEG, _BB), lambda i: (0, i)),
        out_shape=jax.ShapeDtypeStruct((NEG, B), jnp.float32),
        interpret=interpret,
    )(idx8, headT, tailT, tabT)


def _sc_gather(bt_flat, bh_flat, idx_bt2d, idx_bh):
    n_rows, chunk = idx_bt2d.shape
    B = idx_bh.shape[0]
    rows_w = B // _NW
    ck_w = n_rows // _NW
    n_w = ck_w * chunk
    mesh = plsc.VectorSubcoreMesh(core_axis_name="c", subcore_axis_name="s")

    @functools.partial(
        pl.kernel,
        mesh=mesh,
        out_type=(jax.ShapeDtypeStruct((B * 64,), jnp.float32),
                  jax.ShapeDtypeStruct((B,), jnp.float32)),
        scratch_types=[
            pltpu.VMEM((ck_w, chunk), jnp.int32),
            pltpu.VMEM((n_w,), jnp.float32),
            pltpu.VMEM((rows_w,), jnp.int32),
            pltpu.VMEM((rows_w,), jnp.float32),
            [pltpu.SemaphoreType.DMA] * _NSEM,
            pltpu.SemaphoreType.DMA,
        ],
    )
    def run(bt_h, bh_h, ibt_h, ibh_h, out_h, outh_h,
            ibt_v, vbt_v, ibh_v, vbh_v, sems, semh):
        wid = lax.axis_index("s") * 2 + lax.axis_index("c")
        pltpu.sync_copy(ibt_h.at[pl.ds(wid * ck_w, ck_w)], ibt_v)
        pltpu.sync_copy(ibh_h.at[pl.ds(wid * rows_w, rows_w)], ibh_v)
        pltpu.async_copy(bh_h.at[ibh_v], vbh_v, semh)

        def fire(k, carry):
            off = pl.multiple_of(k * chunk, chunk)
            dst = vbt_v.at[pl.ds(off, chunk)]
            for s in range(_NSEM):
                @pl.when(k % _NSEM == s)
                def _():
                    pltpu.async_copy(bt_h.at[ibt_v.at[k]], dst, sems[s])
            return carry

        lax.fori_loop(0, ck_w, fire, 0)

        pltpu.make_async_copy(bh_h.at[ibh_v], vbh_v, semh).wait()
        pltpu.sync_copy(vbh_v, outh_h.at[pl.ds(wid * rows_w, rows_w)])

        def drain(k, carry):
            off = pl.multiple_of(k * chunk, chunk)
            dst = vbt_v.at[pl.ds(off, chunk)]
            for s in range(_NSEM):
                @pl.when(k % _NSEM == s)
                def _():
                    pltpu.make_async_copy(bt_h.at[ibt_v.at[k]], dst, sems[s]).wait()
            return carry

        lax.fori_loop(0, ck_w, drain, 0)
        pltpu.sync_copy(vbt_v, out_h.at[pl.ds(wid * n_w, n_w)])

    return run(bt_flat, bh_flat, idx_bt2d, idx_bh)


def kernel(head, relation, tail, positive_sample, negative_sample, bh, bt,
           rel_diag, rel, c_param):
    del relation
    B, NEG, D = tail.shape
    nrel = rel.shape[0]
    f32 = jnp.float32

    headT = jnp.transpose(head.reshape(B, D).astype(f32))
    tailT = jnp.transpose(tail.astype(f32), (1, 2, 0))
    rel_idx = positive_sample[:, 1].astype(jnp.int32).reshape(1, B)
    idx8 = jnp.broadcast_to(rel_idx, (8, B))

    tab = jnp.concatenate(
        [rel[:, :D].astype(f32), rel_diag.astype(f32), c_param.astype(f32),
         jnp.zeros((nrel, D - 1), f32)], axis=1)
    tabT = jnp.transpose(jnp.pad(tab, ((0, _KPAD - nrel), (0, 0))))

    idx_bt = jnp.concatenate(
        [positive_sample[:, 2:3], negative_sample], axis=1).astype(jnp.int32)
    idx_bt2d = idx_bt.reshape(-1, 128)
    idx_bh = positive_sample[:, 0].astype(jnp.int32)

    bias_bt, bias_bh = _sc_gather(bt.reshape(-1).astype(f32),
                                  bh.reshape(-1).astype(f32),
                                  idx_bt2d, idx_bh)
    scores = _tc_scores(idx8, headT, tailT, tabT)
    out = (jnp.transpose(scores) + bias_bh.reshape(B, 1)).reshape(B * NEG, 1)
    return out + bias_bt.reshape(B * NEG, 1)
